# baseline (speedup 1.0000x reference)
import functools
import jax
import jax.numpy as jnp
from jax import lax
from jax.experimental import pallas as pl
from jax.experimental.pallas import tpu as pltpu, tpu_sc as plsc

_B, _S, _D = 4, 8192, 1024
_NC, _NS, _L = 2, 16, 16
_NW = _NC * _NS
_ROWS_PER_W = _S // _NW
_R = 32
_STEPS = _ROWS_PER_W // _R
_CHUNK = _R * _D


def kernel(x, table):
    mesh = plsc.VectorSubcoreMesh(core_axis_name="c", subcore_axis_name="s")

    @functools.partial(
        pl.kernel,
        out_type=jax.ShapeDtypeStruct((_B * _S * _D,), jnp.float32),
        mesh=mesh,
        scratch_types=[
            pltpu.VMEM((_CHUNK,), jnp.float32),
            pltpu.VMEM((_CHUNK,), jnp.float32),
        ],
    )
    def k(x_hbm, t_hbm, out_hbm, tbuf, xbuf):
        wid = lax.axis_index("s") * _NC + lax.axis_index("c")
        for s in range(_STEPS):
            trow0 = wid * _ROWS_PER_W + s * _R
            pltpu.sync_copy(t_hbm.at[pl.ds(trow0 * _D, _CHUNK)], tbuf)
            for b in range(_B):
                xoff = (b * _S + trow0) * _D
                pltpu.sync_copy(x_hbm.at[pl.ds(xoff, _CHUNK)], xbuf)

                @pl.loop(0, _CHUNK // _L, unroll=8)
                def _(i):
                    sl = pl.ds(i * _L, _L)
                    xbuf[sl] = xbuf[sl] + tbuf[sl]

                pltpu.sync_copy(xbuf, out_hbm.at[pl.ds(xoff, _CHUNK)])

    out = k(x.reshape(-1), table.reshape(-1))
    return out.reshape(_B, _S, _D)


# speedup vs baseline: 1.1207x; 1.1207x over previous
import functools
import jax
import jax.numpy as jnp
from jax import lax
from jax.experimental import pallas as pl
from jax.experimental.pallas import tpu as pltpu, tpu_sc as plsc

_B, _S, _D = 4, 8192, 1024
_NC, _NS, _L = 2, 16, 16
_NW = _NC * _NS
_ROWS_PER_W = _S // _NW
_R = 16
_STEPS = _ROWS_PER_W // _R
_CHUNK = _R * _D
_NX, _NO, _NT = 3, 2, 2
_NIT = _STEPS * _B


def kernel(x, table):
    mesh = plsc.VectorSubcoreMesh(core_axis_name="c", subcore_axis_name="s")

    scratch = (
        [pltpu.VMEM((_CHUNK,), jnp.float32) for _ in range(_NX + _NO + _NT)]
        + [pltpu.SemaphoreType.DMA for _ in range(_NX + _NO + _NT)]
    )

    @functools.partial(
        pl.kernel,
        out_type=jax.ShapeDtypeStruct((_B * _S * _D,), jnp.float32),
        mesh=mesh,
        scratch_types=scratch,
    )
    def k(x_hbm, t_hbm, out_hbm, x0, x1, x2, o0, o1, t0, t1,
          sx0, sx1, sx2, so0, so1, st0, st1):
        xbuf, obuf, tbuf = [x0, x1, x2], [o0, o1], [t0, t1]
        sx, so, st = [sx0, sx1, sx2], [so0, so1], [st0, st1]
        wid = lax.axis_index("s") * _NC + lax.axis_index("c")
        base = wid * _ROWS_PER_W

        def xoff(it):
            s, b = it // _B, it % _B
            return ((b * _S) + base + s * _R) * _D

        def start_in(it):
            return pltpu.async_copy(
                x_hbm.at[pl.ds(xoff(it), _CHUNK)], xbuf[it % _NX], sx[it % _NX])

        def start_t(s):
            return pltpu.async_copy(
                t_hbm.at[pl.ds((base + s * _R) * _D, _CHUNK)],
                tbuf[s % _NT], st[s % _NT])

        ht = {0: start_t(0)}
        hin = {it: start_in(it) for it in range(_NX)}
        hout = {}

        for it in range(_NIT):
            s, b = it // _B, it % _B
            jx, jo = it % _NX, it % _NO
            if b == 0:
                ht.pop(s).wait()
                if s + 1 < _STEPS:
                    ht[s + 1] = start_t(s + 1)
            hin.pop(it).wait()
            if it >= _NO:
                hout.pop(it - _NO).wait()

            tb, xb, ob = tbuf[s % _NT], xbuf[jx], obuf[jo]

            @pl.loop(0, _CHUNK // _L, unroll=8)
            def _(i):
                sl = pl.ds(i * _L, _L)
                ob[sl] = xb[sl] + tb[sl]

            hout[it] = pltpu.async_copy(
                ob, out_hbm.at[pl.ds(xoff(it), _CHUNK)], so[jo])
            if it + _NX < _NIT:
                hin[it + _NX] = start_in(it + _NX)

        for h in hout.values():
            h.wait()

    out = k(x.reshape(-1), table.reshape(-1))
    return out.reshape(_B, _S, _D)


# speedup vs baseline: 1.8162x; 1.6207x over previous
import functools
import jax
import jax.numpy as jnp
from jax import lax
from jax.experimental import pallas as pl
from jax.experimental.pallas import tpu as pltpu, tpu_sc as plsc

_B, _S, _D = 4, 8192, 1024
_NC, _NS, _L = 2, 16, 16
_NW = _NC * _NS
_ROWS_PER_W = _S // _NW
_R = 16
_STEPS = _ROWS_PER_W // _R
_CHUNK = _R * _D
_NX, _NO, _NT = 3, 2, 2
_NIT = _STEPS * _B


def kernel(x, table):
    mesh = plsc.VectorSubcoreMesh(core_axis_name="c", subcore_axis_name="s")

    scratch = (
        [pltpu.VMEM((_CHUNK,), jnp.float32) for _ in range(_NX + _NO + _NT)]
        + [pltpu.SemaphoreType.DMA for _ in range(_NX + _NO + _NT)]
    )

    @functools.partial(
        pl.kernel,
        out_type=jax.ShapeDtypeStruct((_B * _S * _D,), jnp.float32),
        mesh=mesh,
        scratch_types=scratch,
    )
    def k(x_hbm, t_hbm, out_hbm, x0, x1, x2, o0, o1, t0, t1,
          sx0, sx1, sx2, so0, so1, st0, st1):
        xbuf, obuf, tbuf = [x0, x1, x2], [o0, o1], [t0, t1]
        sx, so, st = [sx0, sx1, sx2], [so0, so1], [st0, st1]
        wid = lax.axis_index("s") * _NC + lax.axis_index("c")
        base = wid * _ROWS_PER_W

        def xoff(it):
            s, b = it // _B, it % _B
            return ((b * _S) + base + s * _R) * _D

        def start_in(it):
            return pltpu.async_copy(
                x_hbm.at[pl.ds(xoff(it), _CHUNK)], xbuf[it % _NX], sx[it % _NX])

        def start_t(s):
            return pltpu.async_copy(
                t_hbm.at[pl.ds((base + s * _R) * _D, _CHUNK)],
                tbuf[s % _NT], st[s % _NT])

        ht = {0: start_t(0)}
        hin = {it: start_in(it) for it in range(_NX)}
        hout = {}

        for it in range(_NIT):
            s, b = it // _B, it % _B
            jx, jo = it % _NX, it % _NO
            if b == 0:
                ht.pop(s).wait()
                if s + 1 < _STEPS:
                    ht[s + 1] = start_t(s + 1)
            hin.pop(it).wait()
            if it >= _NO:
                hout.pop(it - _NO).wait()

            tb, xb, ob = tbuf[s % _NT], xbuf[jx], obuf[jo]

            @plsc.parallel_loop(0, _CHUNK // _L, unroll=8)
            def _(i):
                sl = pl.ds(i * _L, _L)
                ob[sl] = xb[sl] + tb[sl]

            hout[it] = pltpu.async_copy(
                ob, out_hbm.at[pl.ds(xoff(it), _CHUNK)], so[jo])
            if it + _NX < _NIT:
                hin[it + _NX] = start_in(it + _NX)

        for h in hout.values():
            h.wait()

    out = k(x.reshape(-1), table.reshape(-1))
    return out.reshape(_B, _S, _D)


# speedup vs baseline: 5.4530x; 3.0024x over previous
import functools
import jax
import jax.numpy as jnp
from jax import lax
from jax.experimental import pallas as pl
from jax.experimental.pallas import tpu as pltpu, tpu_sc as plsc

_B, _S, _D = 4, 8192, 1024
_NC, _NS, _L = 2, 16, 16
_NW = _NC * _NS
_ROWS_PER_W = _S // _NW
_R = 16
_STEPS = _ROWS_PER_W // _R
_GPR = _D // _L
_NX, _NO, _NT = 3, 2, 2
_NIT = _STEPS * _B


def kernel(x, table):
    mesh = plsc.VectorSubcoreMesh(core_axis_name="c", subcore_axis_name="s")

    scratch = (
        [pltpu.VMEM((_R, _D), jnp.float32) for _ in range(_NX + _NO + _NT)]
        + [pltpu.SemaphoreType.DMA for _ in range(_NX + _NO + _NT)]
    )

    @functools.partial(
        pl.kernel,
        out_type=jax.ShapeDtypeStruct((_B, _S, _D), jnp.float32),
        mesh=mesh,
        scratch_types=scratch,
    )
    def k(x_hbm, t_hbm, out_hbm, x0, x1, x2, o0, o1, t0, t1,
          sx0, sx1, sx2, so0, so1, st0, st1):
        xbuf, obuf, tbuf = [x0, x1, x2], [o0, o1], [t0, t1]
        sx, so, st = [sx0, sx1, sx2], [so0, so1], [st0, st1]
        wid = lax.axis_index("s") * _NC + lax.axis_index("c")
        base = wid * _ROWS_PER_W

        def rows(it):
            s, b = it // _B, it % _B
            return b, pl.ds(base + s * _R, _R)

        def start_in(it):
            b, sl = rows(it)
            return pltpu.async_copy(x_hbm.at[b, sl], xbuf[it % _NX], sx[it % _NX])

        def start_t(s):
            return pltpu.async_copy(
                t_hbm.at[pl.ds(base + s * _R, _R)], tbuf[s % _NT], st[s % _NT])

        ht = {0: start_t(0)}
        hin = {it: start_in(it) for it in range(_NX)}
        hout = {}

        for it in range(_NIT):
            s, b = it // _B, it % _B
            jx, jo = it % _NX, it % _NO
            if b == 0:
                ht.pop(s).wait()
                if s + 1 < _STEPS:
                    ht[s + 1] = start_t(s + 1)
            hin.pop(it).wait()
            if it >= _NO:
                hout.pop(it - _NO).wait()

            tb, xb, ob = tbuf[s % _NT], xbuf[jx], obuf[jo]

            @plsc.parallel_loop(0, _R * _GPR, unroll=8)
            def _(i):
                r = lax.shift_right_logical(i, 6)
                c = pl.multiple_of(
                    lax.shift_left(lax.bitwise_and(i, _GPR - 1), 4), _L)
                sl = pl.ds(c, _L)
                ob[r, sl] = xb[r, sl] + tb[r, sl]

            bo, slo = rows(it)
            hout[it] = pltpu.async_copy(ob, out_hbm.at[bo, slo], so[jo])
            if it + _NX < _NIT:
                hin[it + _NX] = start_in(it + _NX)

        for h in hout.values():
            h.wait()

    return k(x, table)
